# Initial kernel scaffold; baseline (speedup 1.0000x reference)
import functools

import jax
import jax.numpy as jnp
from jax import lax
from jax.experimental import pallas as pl
from jax.experimental.pallas import tpu as pltpu
from jax.experimental.pallas import tpu_sc as plsc

N = 10000
E = 10000
NNZ = 320000
DF = 128
WAUG = 144

NCORES = 2
NSUB = 16
NW = NCORES * NSUB
CHUNK = 128
NNZ_T = 10240
NCHUNK = NNZ_T // CHUNK
NNZ_PAD = NW * NNZ_T
ACC_ROWS = 10240
JUNK = N
ROWS_T = ACC_ROWS // NSUB


def _sc_pass_body(gidx_hbm, sidx_hbm, src_hbm, out_hbm,
                  gidx_v, sidx_v, buf0, buf1, acc, sem0, sem1):
    cid = lax.axis_index("c")
    sid = lax.axis_index("s")
    wid = cid * NSUB + sid

    pltpu.sync_copy(gidx_hbm.at[wid], gidx_v)
    pltpu.sync_copy(sidx_hbm.at[wid], sidx_v)

    def _zrow(i, c):
        def _zcol(g, c2):
            buf0[i, pl.ds(g * 16, 16)] = jnp.zeros((16,), jnp.float32)
            return c2
        return lax.fori_loop(0, WAUG // 16, _zcol, c)
    lax.fori_loop(0, CHUNK, _zrow, 0)

    def _zacc(j, c):
        pltpu.sync_copy(buf0, acc.at[pl.ds(sid * ROWS_T + j * CHUNK, CHUNK)])
        return c
    lax.fori_loop(0, ROWS_T // CHUNK, _zacc, 0)
    plsc.subcore_barrier()

    pltpu.async_copy(src_hbm.at[gidx_v.at[0]], buf0, sem0)

    def _body(i, c):
        j = 2 * i
        pltpu.make_async_copy(src_hbm.at[gidx_v.at[0]], buf0, sem0).wait()
        pltpu.async_copy(src_hbm.at[gidx_v.at[j + 1]], buf1, sem1)
        pltpu.sync_copy(buf0, acc.at[sidx_v.at[j]], add=True)
        pltpu.make_async_copy(src_hbm.at[gidx_v.at[0]], buf1, sem1).wait()
        jn = jnp.minimum(j + 2, NCHUNK - 1)
        pltpu.async_copy(src_hbm.at[gidx_v.at[jn]], buf0, sem0)
        pltpu.sync_copy(buf1, acc.at[sidx_v.at[j + 1]], add=True)
        return c
    lax.fori_loop(0, NCHUNK // 2, _body, 0)
    pltpu.make_async_copy(src_hbm.at[gidx_v.at[0]], buf0, sem0).wait()
    plsc.subcore_barrier()

    def _wout(j, c):
        r = sid * ROWS_T + j * CHUNK
        pltpu.sync_copy(acc.at[pl.ds(r, CHUNK)],
                        out_hbm.at[cid, pl.ds(r, CHUNK)])
        return c
    lax.fori_loop(0, ROWS_T // CHUNK, _wout, 0)


_sc_pass = functools.partial(
    pl.kernel,
    out_type=jax.ShapeDtypeStruct((NCORES, ACC_ROWS, WAUG), jnp.float32),
    mesh=plsc.VectorSubcoreMesh(core_axis_name="c", subcore_axis_name="s"),
    scratch_types=[
        pltpu.VMEM((NCHUNK, CHUNK), jnp.int32),
        pltpu.VMEM((NCHUNK, CHUNK), jnp.int32),
        pltpu.VMEM((CHUNK, WAUG), jnp.float32),
        pltpu.VMEM((CHUNK, WAUG), jnp.float32),
        pltpu.VMEM_SHARED((ACC_ROWS, WAUG), jnp.float32),
        pltpu.SemaphoreType.DMA,
        pltpu.SemaphoreType.DMA,
    ],
)(_sc_pass_body)


_MMBLK = 2000


def _mm_aug_kernel(x_ref, w_ref, b_ref, o_ref):
    acc = jnp.dot(x_ref[...], w_ref[...],
                  preferred_element_type=jnp.float32) + b_ref[...]
    o_ref[:, :DF] = acc
    lane = lax.broadcasted_iota(jnp.int32, (_MMBLK, WAUG - DF), 1)
    o_ref[:, DF:WAUG] = jnp.where(lane == 0, 1.0, 0.0).astype(jnp.float32)


def _mm_aug(x, W1, b1):
    return pl.pallas_call(
        _mm_aug_kernel,
        grid=(N // _MMBLK,),
        in_specs=[pl.BlockSpec((_MMBLK, DF), lambda i: (i, 0)),
                  pl.BlockSpec((DF, DF), lambda i: (0, 0)),
                  pl.BlockSpec((1, DF), lambda i: (0, 0))],
        out_specs=pl.BlockSpec((_MMBLK, WAUG), lambda i: (i, 0)),
        out_shape=jax.ShapeDtypeStruct((N, WAUG), jnp.float32),
    )(x, W1, b1)


def _combine_kernel(p_ref, w_ref, o_ref):
    s = p_ref[0] + p_ref[1]
    esum = s[:, :DF]
    cnt = s[:, DF:DF + 1]
    binv = jnp.where(cnt > 0, 1.0 / cnt, 0.0)
    w = w_ref[...]
    o_ref[:, :DF] = esum * (binv * w)
    lane = lax.broadcasted_iota(jnp.int32, (_MMBLK, WAUG - DF), 1)
    o_ref[:, DF:WAUG] = jnp.where(lane == 0, w, 0.0)


def _combine(p, w2d):
    return pl.pallas_call(
        _combine_kernel,
        grid=(E // _MMBLK,),
        in_specs=[pl.BlockSpec((NCORES, _MMBLK, WAUG), lambda i: (0, i, 0)),
                  pl.BlockSpec((_MMBLK, 1), lambda i: (i, 0))],
        out_specs=pl.BlockSpec((_MMBLK, WAUG), lambda i: (i, 0)),
        out_shape=jax.ShapeDtypeStruct((E, WAUG), jnp.float32),
    )(p, w2d)


def _final_kernel(p_ref, x_ref, g_ref, b_ref, o_ref):
    s = p_ref[0] + p_ref[1]
    osum = s[:, :DF]
    d = s[:, DF:DF + 1]
    dinv = jnp.where(d > 0, 1.0 / d, 0.0)
    h = jnp.maximum(x_ref[...] + osum * dinv, 0.0)
    mu = jnp.mean(h, axis=1, keepdims=True)
    var = jnp.mean((h - mu) ** 2, axis=1, keepdims=True)
    o_ref[...] = (h - mu) * lax.rsqrt(var + 1e-5) * g_ref[...] + b_ref[...]


def _final(p, x, gamma, beta):
    return pl.pallas_call(
        _final_kernel,
        grid=(N // _MMBLK,),
        in_specs=[pl.BlockSpec((NCORES, _MMBLK, WAUG), lambda i: (0, i, 0)),
                  pl.BlockSpec((_MMBLK, DF), lambda i: (i, 0)),
                  pl.BlockSpec((1, DF), lambda i: (0, 0)),
                  pl.BlockSpec((1, DF), lambda i: (0, 0))],
        out_specs=pl.BlockSpec((_MMBLK, DF), lambda i: (i, 0)),
        out_shape=jax.ShapeDtypeStruct((N, DF), jnp.float32),
    )(p, x, gamma, beta)


def kernel(x, hyperedge_index, hyperedge_weight, W1, b1, gamma, beta):
    node = hyperedge_index[0]
    edge = hyperedge_index[1]
    pad = NNZ_PAD - NNZ
    padg = jnp.zeros((pad,), jnp.int32)
    pads = jnp.full((pad,), JUNK, jnp.int32)
    g1 = jnp.concatenate([node, padg]).reshape(NW, NCHUNK, CHUNK)
    s1 = jnp.concatenate([edge, pads]).reshape(NW, NCHUNK, CHUNK)
    g2 = jnp.concatenate([edge, padg]).reshape(NW, NCHUNK, CHUNK)
    s2 = jnp.concatenate([node, pads]).reshape(NW, NCHUNK, CHUNK)

    xt_aug = _mm_aug(x, W1, b1.reshape(1, DF))
    p1 = _sc_pass(g1, s1, xt_aug)
    e_aug = _combine(p1, hyperedge_weight.reshape(E, 1))
    p2 = _sc_pass(g2, s2, e_aug)
    return _final(p2, x, gamma.reshape(1, DF), beta.reshape(1, DF))


# speedup vs baseline: 4.8728x; 4.8728x over previous
import functools

import jax
import jax.numpy as jnp
from jax import lax
from jax.experimental import pallas as pl
from jax.experimental.pallas import tpu as pltpu
from jax.experimental.pallas import tpu_sc as plsc

N = 10000
E = 10000
NNZ = 320000
DF = 128

NCORES = 2
NSUB = 16
NW = NCORES * NSUB
CHUNK = 128
NNZ_T = 10240
NCHUNK = NNZ_T // CHUNK
NHALF = NCHUNK // 2
NNZ_PAD = NW * NNZ_T
SEG = 10240
JUNK = N
SEG_T = SEG // NSUB


def _sc_pass_body(gidx_hbm, sidx_hbm, src_hbm, svec_hbm, out_hbm, out2_hbm,
                  gidx_v, sidx_v, buf0, buf1, val0, val1, zbuf,
                  acc, sacc, sem0, sem1, sem2, sem3):
    cid = lax.axis_index("c")
    sid = lax.axis_index("s")
    wid = cid * NSUB + sid

    def _zrow(i, c):
        def _zcol(g, c2):
            buf0[i, pl.ds(g * 16, 16)] = jnp.zeros((16,), jnp.float32)
            return c2
        return lax.fori_loop(0, DF // 16, _zcol, c)
    lax.fori_loop(0, CHUNK, _zrow, 0)

    def _zb(g, c):
        zbuf[pl.ds(g * 16, 16)] = jnp.zeros((16,), jnp.float32)
        return c
    lax.fori_loop(0, SEG_T // 16, _zb, 0)

    def _zacc(j, c):
        pltpu.sync_copy(buf0, acc.at[pl.ds(sid * SEG_T + j * CHUNK, CHUNK)])
        return c
    lax.fori_loop(0, SEG_T // CHUNK, _zacc, 0)
    pltpu.sync_copy(zbuf, sacc.at[pl.ds(sid * SEG_T, SEG_T)])
    plsc.subcore_barrier()

    def _issue(j, buf, val, semr, sems):
        pltpu.async_copy(src_hbm.at[gidx_v.at[j]], buf, semr)
        pltpu.async_copy(svec_hbm.at[gidx_v.at[j]], val, sems)

    def _wait(buf, val, semr, sems):
        pltpu.make_async_copy(src_hbm.at[gidx_v.at[0]], buf, semr).wait()
        pltpu.make_async_copy(svec_hbm.at[gidx_v.at[0]], val, sems).wait()

    def _scatter(j, buf, val):
        pltpu.sync_copy(buf, acc.at[sidx_v.at[j]], add=True)
        pltpu.sync_copy(val, sacc.at[sidx_v.at[j]], add=True)

    for h in range(NCHUNK // NHALF):
        pltpu.sync_copy(gidx_hbm.at[wid, pl.ds(h * NHALF, NHALF)], gidx_v)
        pltpu.sync_copy(sidx_hbm.at[wid, pl.ds(h * NHALF, NHALF)], sidx_v)
        _issue(0, buf0, val0, sem0, sem2)

        def _body(i, c):
            j = 2 * i
            _wait(buf0, val0, sem0, sem2)
            _issue(j + 1, buf1, val1, sem1, sem3)
            _scatter(j, buf0, val0)
            _wait(buf1, val1, sem1, sem3)
            jn = jnp.minimum(j + 2, NHALF - 1)
            _issue(jn, buf0, val0, sem0, sem2)
            _scatter(j + 1, buf1, val1)
            return c
        lax.fori_loop(0, NHALF // 2, _body, 0)
        _wait(buf0, val0, sem0, sem2)
    plsc.subcore_barrier()

    def _wout(j, c):
        r = sid * SEG_T + j * CHUNK
        pltpu.sync_copy(acc.at[pl.ds(r, CHUNK)],
                        out_hbm.at[cid, pl.ds(r, CHUNK)])
        return c
    lax.fori_loop(0, SEG_T // CHUNK, _wout, 0)
    pltpu.sync_copy(sacc.at[pl.ds(sid * SEG_T, SEG_T)],
                    out2_hbm.at[cid, pl.ds(sid * SEG_T, SEG_T)])


_sc_pass = functools.partial(
    pl.kernel,
    out_type=(jax.ShapeDtypeStruct((NCORES, SEG, DF), jnp.float32),
              jax.ShapeDtypeStruct((NCORES, SEG), jnp.float32)),
    mesh=plsc.VectorSubcoreMesh(core_axis_name="c", subcore_axis_name="s"),
    compiler_params=pltpu.CompilerParams(needs_layout_passes=False),
    scratch_types=[
        pltpu.VMEM((NHALF, CHUNK), jnp.int32),
        pltpu.VMEM((NHALF, CHUNK), jnp.int32),
        pltpu.VMEM((CHUNK, DF), jnp.float32),
        pltpu.VMEM((CHUNK, DF), jnp.float32),
        pltpu.VMEM((CHUNK,), jnp.float32),
        pltpu.VMEM((CHUNK,), jnp.float32),
        pltpu.VMEM((SEG_T,), jnp.float32),
        pltpu.VMEM_SHARED((SEG, DF), jnp.float32),
        pltpu.VMEM_SHARED((SEG,), jnp.float32),
        pltpu.SemaphoreType.DMA,
        pltpu.SemaphoreType.DMA,
        pltpu.SemaphoreType.DMA,
        pltpu.SemaphoreType.DMA,
    ],
)(_sc_pass_body)


_MMBLK = 2000
_CBLK = 2048


def _mm_kernel(x_ref, w_ref, b_ref, o_ref):
    o_ref[...] = jnp.dot(x_ref[...], w_ref[...],
                         preferred_element_type=jnp.float32) + b_ref[...]


def _mm(x, W1, b1):
    return pl.pallas_call(
        _mm_kernel,
        grid=(N // _MMBLK,),
        in_specs=[pl.BlockSpec((_MMBLK, DF), lambda i: (i, 0)),
                  pl.BlockSpec((DF, DF), lambda i: (0, 0)),
                  pl.BlockSpec((1, DF), lambda i: (0, 0))],
        out_specs=pl.BlockSpec((_MMBLK, DF), lambda i: (i, 0)),
        out_shape=jax.ShapeDtypeStruct((N, DF), jnp.float32),
    )(x, W1, b1)


def _col_from_tile(s16):
    t = lax.broadcast_in_dim(s16, (16, 128, 128), (0, 2))
    t2 = t.reshape(_CBLK, 128)
    lane = lax.broadcasted_iota(jnp.int32, (_CBLK, 128), 1)
    row = lax.broadcasted_iota(jnp.int32, (_CBLK, 128), 0)
    return jnp.sum(jnp.where(lane == row % 128, t2, 0.0), axis=1,
                   keepdims=True)


def _combine_kernel(p_ref, q_ref, w_ref, o_ref):
    esum = p_ref[0] + p_ref[1]
    cnt = _col_from_tile(q_ref[0] + q_ref[1])
    binv = jnp.where(cnt > 0, 1.0 / cnt, 0.0)
    o_ref[...] = esum * (binv * w_ref[...])


def _combine(p, q3, w2d):
    return pl.pallas_call(
        _combine_kernel,
        grid=(SEG // _CBLK,),
        in_specs=[pl.BlockSpec((NCORES, _CBLK, DF), lambda i: (0, i, 0)),
                  pl.BlockSpec((NCORES, _CBLK // 128, 128),
                               lambda i: (0, i, 0)),
                  pl.BlockSpec((_CBLK, 1), lambda i: (i, 0))],
        out_specs=pl.BlockSpec((_CBLK, DF), lambda i: (i, 0)),
        out_shape=jax.ShapeDtypeStruct((E, DF), jnp.float32),
    )(p, q3, w2d)


def _final_kernel(p_ref, q_ref, x_ref, g_ref, b_ref, o_ref):
    osum = p_ref[0] + p_ref[1]
    d = _col_from_tile(q_ref[0] + q_ref[1])
    dinv = jnp.where(d > 0, 1.0 / d, 0.0)
    h = jnp.maximum(x_ref[...] + osum * dinv, 0.0)
    mu = jnp.mean(h, axis=1, keepdims=True)
    var = jnp.mean((h - mu) ** 2, axis=1, keepdims=True)
    o_ref[...] = (h - mu) * lax.rsqrt(var + 1e-5) * g_ref[...] + b_ref[...]


def _final(p, q3, x, gamma, beta):
    return pl.pallas_call(
        _final_kernel,
        grid=(SEG // _CBLK,),
        in_specs=[pl.BlockSpec((NCORES, _CBLK, DF), lambda i: (0, i, 0)),
                  pl.BlockSpec((NCORES, _CBLK // 128, 128),
                               lambda i: (0, i, 0)),
                  pl.BlockSpec((_CBLK, DF), lambda i: (i, 0)),
                  pl.BlockSpec((1, DF), lambda i: (0, 0)),
                  pl.BlockSpec((1, DF), lambda i: (0, 0))],
        out_specs=pl.BlockSpec((_CBLK, DF), lambda i: (i, 0)),
        out_shape=jax.ShapeDtypeStruct((N, DF), jnp.float32),
    )(p, q3, x, gamma, beta)


def kernel(x, hyperedge_index, hyperedge_weight, W1, b1, gamma, beta):
    node = hyperedge_index[0]
    edge = hyperedge_index[1]
    pad = NNZ_PAD - NNZ
    padg = jnp.zeros((pad,), jnp.int32)
    pads = jnp.full((pad,), JUNK, jnp.int32)
    g1 = jnp.concatenate([node, padg]).reshape(NW, NCHUNK, CHUNK)
    s1 = jnp.concatenate([edge, pads]).reshape(NW, NCHUNK, CHUNK)
    g2 = jnp.concatenate([edge, padg]).reshape(NW, NCHUNK, CHUNK)
    s2 = jnp.concatenate([node, pads]).reshape(NW, NCHUNK, CHUNK)
    ones = jnp.ones((SEG,), jnp.float32)
    wpad = jnp.concatenate([hyperedge_weight,
                            jnp.zeros((SEG - E,), jnp.float32)])

    xt = _mm(x, W1, b1.reshape(1, DF))
    p1, q1 = _sc_pass(g1, s1, xt, ones)
    e = _combine(p1, q1.reshape(NCORES, SEG // 128, 128),
                 hyperedge_weight.reshape(E, 1))
    p2, q2 = _sc_pass(g2, s2, e, wpad)
    return _final(p2, q2.reshape(NCORES, SEG // 128, 128),
                  x, gamma.reshape(1, DF), beta.reshape(1, DF))
